# Initial kernel scaffold; baseline (speedup 1.0000x reference)
import jax
import jax.numpy as jnp
from jax.experimental import pallas as pl


def kernel(x, edge_index, edge_attr, u, eb_W, eb_b, nb_W, nb_b, gb_W, gb_b, dec_W1, dec_b1, dec_W2, dec_b2):
    raise NotImplementedError("write your pallas kernel here")


# speedup vs baseline: 2.9069x; 2.9069x over previous
import functools

import jax
import jax.numpy as jnp
from jax import lax
from jax.experimental import pallas as pl
from jax.experimental.pallas import tpu as pltpu
from jax.experimental.pallas import tpu_sc as plsc

F32 = jnp.float32

NC = 2
NS = 16
NW = NC * NS
C = 80
RB = 80


def _node_proj(x, Ws, Wr, bn=1000):
    N, D = x.shape
    H = Ws.shape[1]

    def body(x_ref, ws_ref, wr_ref, xs_ref, xr_ref):
        xb = x_ref[...]
        xs_ref[...] = jnp.dot(xb, ws_ref[...], preferred_element_type=F32)
        xr_ref[...] = jnp.dot(xb, wr_ref[...], preferred_element_type=F32)

    return pl.pallas_call(
        body,
        grid=(N // bn,),
        in_specs=[
            pl.BlockSpec((bn, D), lambda i: (i, 0)),
            pl.BlockSpec((D, H), lambda i: (0, 0)),
            pl.BlockSpec((D, H), lambda i: (0, 0)),
        ],
        out_specs=[
            pl.BlockSpec((bn, H), lambda i: (i, 0)),
            pl.BlockSpec((bn, H), lambda i: (i, 0)),
        ],
        out_shape=[
            jax.ShapeDtypeStruct((N, H), F32),
            jax.ShapeDtypeStruct((N, H), F32),
        ],
    )(x, Ws, Wr)


def _edge_proj(edge_attr, Wea, u, Wu, eb_b, be=4000):
    E, De = edge_attr.shape
    H = Wea.shape[1]

    def body(ea_ref, wea_ref, u_ref, wu_ref, b_ref, out_ref):
        c1 = jnp.dot(u_ref[...], wu_ref[...], preferred_element_type=F32) + b_ref[...]
        out_ref[...] = (
            jnp.dot(ea_ref[...], wea_ref[...], preferred_element_type=F32) + c1
        )

    return pl.pallas_call(
        body,
        grid=(E // be,),
        in_specs=[
            pl.BlockSpec((be, De), lambda i: (i, 0)),
            pl.BlockSpec((De, H), lambda i: (0, 0)),
            pl.BlockSpec((1, H), lambda i: (0, 0)),
            pl.BlockSpec((H, H), lambda i: (0, 0)),
            pl.BlockSpec((1, H), lambda i: (0, 0)),
        ],
        out_specs=pl.BlockSpec((be, H), lambda i: (i, 0)),
        out_shape=jax.ShapeDtypeStruct((E, H), F32),
    )(edge_attr, Wea, u, Wu, eb_b)


def _node_block(aggP, x, Wagg, Wx, u, Wnu, nb_b, W1s, W1r, bn=1000):
    N, D = x.shape
    H = Wagg.shape[1]

    def body(aggP_ref, x_ref, wagg_ref, wx_ref, u_ref, wnu_ref, b_ref,
             w1s_ref, w1r_ref, ns_ref, nr_ref, nsum_ref, esum_ref):
        agg = aggP_ref[0] + aggP_ref[1]
        cn = jnp.dot(u_ref[...], wnu_ref[...], preferred_element_type=F32) + b_ref[...]
        n1 = jnp.maximum(
            jnp.dot(agg, wagg_ref[...], preferred_element_type=F32)
            + jnp.dot(x_ref[...], wx_ref[...], preferred_element_type=F32)
            + cn, 0.0)
        ns_ref[...] = jnp.dot(n1, w1s_ref[...], preferred_element_type=F32)
        nr_ref[...] = jnp.dot(n1, w1r_ref[...], preferred_element_type=F32)

        @pl.when(pl.program_id(0) == 0)
        def _():
            nsum_ref[...] = jnp.zeros_like(nsum_ref)
            esum_ref[...] = jnp.zeros_like(esum_ref)

        nsum_ref[...] += jnp.sum(n1, axis=0, keepdims=True)
        esum_ref[...] += jnp.sum(agg, axis=0, keepdims=True)

    return pl.pallas_call(
        body,
        grid=(N // bn,),
        in_specs=[
            pl.BlockSpec((NC, bn, H), lambda i: (0, i, 0)),
            pl.BlockSpec((bn, D), lambda i: (i, 0)),
            pl.BlockSpec((H, H), lambda i: (0, 0)),
            pl.BlockSpec((D, H), lambda i: (0, 0)),
            pl.BlockSpec((1, H), lambda i: (0, 0)),
            pl.BlockSpec((H, H), lambda i: (0, 0)),
            pl.BlockSpec((1, H), lambda i: (0, 0)),
            pl.BlockSpec((H, H), lambda i: (0, 0)),
            pl.BlockSpec((H, H), lambda i: (0, 0)),
        ],
        out_specs=[
            pl.BlockSpec((bn, H), lambda i: (i, 0)),
            pl.BlockSpec((bn, H), lambda i: (i, 0)),
            pl.BlockSpec((1, H), lambda i: (0, 0)),
            pl.BlockSpec((1, H), lambda i: (0, 0)),
        ],
        out_shape=[
            jax.ShapeDtypeStruct((N, H), F32),
            jax.ShapeDtypeStruct((N, H), F32),
            jax.ShapeDtypeStruct((1, H), F32),
            jax.ShapeDtypeStruct((1, H), F32),
        ],
    )(aggP, x, Wagg, Wx, u, Wnu, nb_b, W1s, W1r)


def _decoder(e1, d, esum, nsum, u, gbW_e, gbW_n, gbW_u, gb_b, W1e, W1g,
             dec_b1, dec_W2, dec_b2, n_edges, n_nodes, be=4000):
    E, H = e1.shape
    OUT = dec_W2.shape[1]

    def body(e1_ref, d_ref, esum_ref, nsum_ref, u_ref, gbe_ref, gbn_ref,
             gbu_ref, gbb_ref, w1e_ref, w1g_ref, b1_ref, w2_ref, b2_ref,
             out_ref):
        g1 = jnp.maximum(
            jnp.dot(esum_ref[...] * (1.0 / n_edges), gbe_ref[...],
                    preferred_element_type=F32)
            + jnp.dot(nsum_ref[...] * (1.0 / n_nodes), gbn_ref[...],
                      preferred_element_type=F32)
            + jnp.dot(u_ref[...], gbu_ref[...], preferred_element_type=F32)
            + gbb_ref[...], 0.0)
        cdec = jnp.dot(g1, w1g_ref[...], preferred_element_type=F32) + b1_ref[...]
        h = jnp.maximum(
            jnp.dot(e1_ref[...], w1e_ref[...], preferred_element_type=F32)
            + d_ref[...] + cdec, 0.0)
        out_ref[...] = jnp.dot(h, w2_ref[...], preferred_element_type=F32) + b2_ref[...]

    return pl.pallas_call(
        body,
        grid=(E // be,),
        in_specs=[
            pl.BlockSpec((be, H), lambda i: (i, 0)),
            pl.BlockSpec((be, H), lambda i: (i, 0)),
            pl.BlockSpec((1, H), lambda i: (0, 0)),
            pl.BlockSpec((1, H), lambda i: (0, 0)),
            pl.BlockSpec((1, H), lambda i: (0, 0)),
            pl.BlockSpec((H, H), lambda i: (0, 0)),
            pl.BlockSpec((H, H), lambda i: (0, 0)),
            pl.BlockSpec((H, H), lambda i: (0, 0)),
            pl.BlockSpec((1, H), lambda i: (0, 0)),
            pl.BlockSpec((H, H), lambda i: (0, 0)),
            pl.BlockSpec((H, H), lambda i: (0, 0)),
            pl.BlockSpec((1, H), lambda i: (0, 0)),
            pl.BlockSpec((H, OUT), lambda i: (0, 0)),
            pl.BlockSpec((1, OUT), lambda i: (0, 0)),
        ],
        out_specs=pl.BlockSpec((be, OUT), lambda i: (i, 0)),
        out_shape=jax.ShapeDtypeStruct((E, OUT), F32),
    )(e1, d, esum, nsum, u, gbW_e, gbW_n, gbW_u, gb_b, W1e, W1g, dec_b1,
      dec_W2, dec_b2)


def _sc_edge(eap, xs, xr, s3d, r3d):
    E, H = eap.shape
    N = xs.shape[0]
    epw = E // NW
    nch = epw // C
    nrb = N // RB
    nzi = (nrb + NS - 1) // NS
    mesh = plsc.VectorSubcoreMesh(core_axis_name="c", subcore_axis_name="s")

    @functools.partial(
        pl.kernel,
        out_type=[
            jax.ShapeDtypeStruct((E, H), F32),
            jax.ShapeDtypeStruct((NC, N, H), F32),
        ],
        mesh=mesh,
        compiler_params=pltpu.CompilerParams(use_tc_tiling_on_sc=False),
        scratch_types=[
            pltpu.VMEM((nch, C), jnp.int32),
            pltpu.VMEM((nch, C), jnp.int32),
            pltpu.VMEM((C, H), F32),
            pltpu.VMEM((C, H), F32),
            pltpu.VMEM((C, H), F32),
            pltpu.VMEM((RB, H), F32),
            pltpu.VMEM_SHARED((N, H), F32),
            pltpu.SemaphoreType.DMA,
            pltpu.SemaphoreType.DMA,
        ],
    )
    def k(eap_hbm, xs_hbm, xr_hbm, s3d_hbm, r3d_hbm, e1_hbm, agg_hbm,
          idx_s, idx_r, bufE, bufS, bufR, zbuf, acc, sem1, sem2):
        cid = lax.axis_index("c")
        sid = lax.axis_index("s")
        wid = sid * NC + cid

        def zrow(i, carry):
            r = i // (H // 16)
            cc = i % (H // 16)
            zbuf[r, pl.ds(cc * 16, 16)] = jnp.zeros((16,), F32)
            return carry

        lax.fori_loop(0, RB * (H // 16), zrow, 0)

        def zacc(i, carry):
            b = sid * nzi + i

            @pl.when(b < nrb)
            def _():
                pltpu.sync_copy(zbuf, acc.at[pl.ds(b * RB, RB)])

            return carry

        lax.fori_loop(0, nzi, zacc, 0)
        plsc.subcore_barrier()

        pltpu.sync_copy(s3d_hbm.at[wid], idx_s)
        pltpu.sync_copy(r3d_hbm.at[wid], idx_r)
        base_e = wid * epw

        def chunk(j, carry):
            e0 = base_e + j * C
            pltpu.sync_copy(eap_hbm.at[pl.ds(e0, C)], bufE)
            cpS = pltpu.async_copy(xs_hbm.at[idx_s.at[j]], bufS, sem1)
            cpR = pltpu.async_copy(xr_hbm.at[idx_r.at[j]], bufR, sem2)
            cpS.wait()
            cpR.wait()

            def row(r, rc):
                for cc in range(H // 16):
                    sl = pl.ds(cc * 16, 16)
                    v = bufE[r, sl] + bufS[r, sl] + bufR[r, sl]
                    bufE[r, sl] = jnp.maximum(v, 0.0)
                return rc

            lax.fori_loop(0, C, row, 0)
            pltpu.sync_copy(bufE, e1_hbm.at[pl.ds(e0, C)])
            pltpu.sync_copy(bufE, acc.at[idx_r.at[j]], add=True)
            return carry

        lax.fori_loop(0, nch, chunk, 0)
        plsc.subcore_barrier()

        def flush(i, carry):
            b = sid * nzi + i

            @pl.when(b < nrb)
            def _():
                pltpu.sync_copy(acc.at[pl.ds(b * RB, RB)], zbuf)
                pltpu.sync_copy(zbuf, agg_hbm.at[cid, pl.ds(b * RB, RB)])

            return carry

        lax.fori_loop(0, nzi, flush, 0)

    return k(eap, xs, xr, s3d, r3d)


def _sc_dec_gather(ns, nr, s3d, r3d, E):
    N, H = ns.shape
    epw = E // NW
    nch = epw // C
    mesh = plsc.VectorSubcoreMesh(core_axis_name="c", subcore_axis_name="s")

    @functools.partial(
        pl.kernel,
        out_type=jax.ShapeDtypeStruct((E, H), F32),
        mesh=mesh,
        compiler_params=pltpu.CompilerParams(use_tc_tiling_on_sc=False),
        scratch_types=[
            pltpu.VMEM((nch, C), jnp.int32),
            pltpu.VMEM((nch, C), jnp.int32),
            pltpu.VMEM((C, H), F32),
            pltpu.VMEM((C, H), F32),
            pltpu.SemaphoreType.DMA,
            pltpu.SemaphoreType.DMA,
        ],
    )
    def k(ns_hbm, nr_hbm, s3d_hbm, r3d_hbm, d_hbm,
          idx_s, idx_r, bufA, bufB, sem1, sem2):
        cid = lax.axis_index("c")
        sid = lax.axis_index("s")
        wid = sid * NC + cid
        pltpu.sync_copy(s3d_hbm.at[wid], idx_s)
        pltpu.sync_copy(r3d_hbm.at[wid], idx_r)
        base_e = wid * epw

        def chunk(j, carry):
            cpA = pltpu.async_copy(ns_hbm.at[idx_s.at[j]], bufA, sem1)
            cpB = pltpu.async_copy(nr_hbm.at[idx_r.at[j]], bufB, sem2)
            cpA.wait()
            cpB.wait()

            def row(r, rc):
                for cc in range(H // 16):
                    sl = pl.ds(cc * 16, 16)
                    bufA[r, sl] = bufA[r, sl] + bufB[r, sl]
                return rc

            lax.fori_loop(0, C, row, 0)
            pltpu.sync_copy(bufA, d_hbm.at[pl.ds(base_e + j * C, C)])
            return carry

        lax.fori_loop(0, nch, chunk, 0)

    return k(ns, nr, s3d, r3d)


def kernel(x, edge_index, edge_attr, u, eb_W, eb_b, nb_W, nb_b, gb_W, gb_b,
           dec_W1, dec_b1, dec_W2, dec_b2):
    N, D = x.shape
    E = edge_index.shape[1]
    De = edge_attr.shape[1]
    H = eb_W.shape[1]
    assert E % (NW * C) == 0 and N % RB == 0

    senders = edge_index[0]
    receivers = edge_index[1]
    s3d = senders.reshape(NW, E // (NW * C), C)
    r3d = receivers.reshape(NW, E // (NW * C), C)

    Wea = eb_W[:De]
    Ws = eb_W[De:De + D]
    Wr = eb_W[De + D:De + 2 * D]
    Wu = eb_W[De + 2 * D:]
    Wagg = nb_W[:H]
    Wx = nb_W[H:H + D]
    Wnu = nb_W[H + D:]
    gbW_e = gb_W[:H]
    gbW_n = gb_W[H:2 * H]
    gbW_u = gb_W[2 * H:]
    W1e = dec_W1[:H]
    W1s = dec_W1[H:2 * H]
    W1r = dec_W1[2 * H:3 * H]
    W1g = dec_W1[3 * H:]

    u2 = u.reshape(1, H)
    eb_b2 = eb_b.reshape(1, H)
    nb_b2 = nb_b.reshape(1, H)
    gb_b2 = gb_b.reshape(1, H)
    dec_b12 = dec_b1.reshape(1, H)
    dec_b22 = dec_b2.reshape(1, -1)

    xs, xr = _node_proj(x, Ws, Wr)
    eap = _edge_proj(edge_attr, Wea, u2, Wu, eb_b2)
    e1, aggP = _sc_edge(eap, xs, xr, s3d, r3d)
    ns, nr, nsum, esum = _node_block(aggP, x, Wagg, Wx, u2, Wnu, nb_b2,
                                     W1s, W1r)
    d = _sc_dec_gather(ns, nr, s3d, r3d, E)
    out = _decoder(e1, d, esum, nsum, u2, gbW_e, gbW_n, gbW_u, gb_b2,
                   W1e, W1g, dec_b12, dec_W2, dec_b22, E, N)
    return out
